# baseline (speedup 1.0000x reference)
import functools

import jax
import jax.numpy as jnp
from jax.experimental import pallas as pl
from jax.experimental.pallas import tpu as pltpu

_BLOCK_ROWS = 2000


def _body(x_ref, t_ref, w0_ref, w1_ref, w2_ref, b_ref, o_ref):
    x = x_ref[:]
    y0 = jnp.dot(x, w0_ref[:], preferred_element_type=jnp.float32)
    y1 = jnp.dot(x, w1_ref[:], preferred_element_type=jnp.float32)
    y2 = jnp.dot(x, w2_ref[:], preferred_element_type=jnp.float32)
    b = b_ref[:]
    t = t_ref[:]
    y0 = y0 + b[0:1, :]
    y1 = y1 + b[1:2, :]
    y2 = y2 + b[2:3, :]
    out = jnp.where(t == 0, y0, jnp.where(t == 1, y1, y2))
    o_ref[:] = jnp.where(t >= 3, 0.0, out)


@functools.partial(jax.jit, static_argnames=("interpret",))
def _run(feats, ntypes, w0, w1, w2, b_all, interpret=False):
    n, d = feats.shape
    p = w2.shape[1]
    r = _BLOCK_ROWS
    grid = (n // r,)
    return pl.pallas_call(
        _body,
        grid=grid,
        in_specs=[
            pl.BlockSpec((r, d), lambda i: (i, 0)),
            pl.BlockSpec((r, 1), lambda i: (i, 0)),
            pl.BlockSpec((d, p), lambda i: (0, 0)),
            pl.BlockSpec((d, p), lambda i: (0, 0)),
            pl.BlockSpec((d, p), lambda i: (0, 0)),
            pl.BlockSpec((3, p), lambda i: (0, 0)),
        ],
        out_specs=pl.BlockSpec((r, p), lambda i: (i, 0)),
        out_shape=jax.ShapeDtypeStruct((n, p), feats.dtype),
        compiler_params=pltpu.CompilerParams(
            dimension_semantics=("arbitrary",),
        ),
        interpret=interpret,
    )(feats, ntypes, w0, w1, w2, b_all)


def kernel(feats, ntypes, W_device, b_device, W_inst, b_inst, W_net, b_net):
    d = feats.shape[1]
    p = W_net.shape[1]
    w0 = jnp.zeros((d, p), W_device.dtype).at[: W_device.shape[0]].set(W_device)
    w1 = jnp.zeros((d, p), W_inst.dtype).at[: W_inst.shape[0]].set(W_inst)
    b_all = jnp.stack([b_device, b_inst, b_net], axis=0)
    t2d = ntypes.reshape(-1, 1)
    return _run(feats, t2d, w0, w1, W_net, b_all)


# speedup vs baseline: 1.0117x; 1.0117x over previous
import functools

import jax
import jax.numpy as jnp
from jax.experimental import pallas as pl
from jax.experimental.pallas import tpu as pltpu

_BLOCK_ROWS = 2000


def _body(x_ref, t_ref, w0_ref, w1_ref, w2_ref, b_ref, o_ref):
    x = x_ref[:]
    d0 = w0_ref.shape[0]
    d1 = w1_ref.shape[0]
    y0 = jnp.dot(x[:, :d0], w0_ref[:], preferred_element_type=jnp.float32)
    y1 = jnp.dot(x[:, :d1], w1_ref[:], preferred_element_type=jnp.float32)
    y2 = jnp.dot(x, w2_ref[:], preferred_element_type=jnp.float32)
    b = b_ref[:]
    t = t_ref[:]
    y0 = y0 + b[0:1, :]
    y1 = y1 + b[1:2, :]
    y2 = y2 + b[2:3, :]
    out = jnp.where(t == 0, y0, jnp.where(t == 1, y1, y2))
    o_ref[:] = jnp.where(t >= 3, 0.0, out)


@functools.partial(jax.jit, static_argnames=("interpret",))
def _run(feats, ntypes, w0, w1, w2, b_all, interpret=False):
    n, d = feats.shape
    p = w2.shape[1]
    r = _BLOCK_ROWS
    grid = (n // r,)
    return pl.pallas_call(
        _body,
        grid=grid,
        in_specs=[
            pl.BlockSpec((r, d), lambda i: (i, 0)),
            pl.BlockSpec((r, 1), lambda i: (i, 0)),
            pl.BlockSpec(w0.shape, lambda i: (0, 0)),
            pl.BlockSpec(w1.shape, lambda i: (0, 0)),
            pl.BlockSpec(w2.shape, lambda i: (0, 0)),
            pl.BlockSpec((3, p), lambda i: (0, 0)),
        ],
        out_specs=pl.BlockSpec((r, p), lambda i: (i, 0)),
        out_shape=jax.ShapeDtypeStruct((n, p), feats.dtype),
        compiler_params=pltpu.CompilerParams(
            dimension_semantics=("arbitrary",),
        ),
        interpret=interpret,
    )(feats, ntypes, w0, w1, w2, b_all)


def kernel(feats, ntypes, W_device, b_device, W_inst, b_inst, W_net, b_net):
    b_all = jnp.stack([b_device, b_inst, b_net], axis=0)
    t2d = ntypes.reshape(-1, 1)
    return _run(feats, t2d, W_device, W_inst, W_net, b_all)


# speedup vs baseline: 1.1098x; 1.0970x over previous
import functools

import jax
import jax.numpy as jnp
from jax.experimental import pallas as pl
from jax.experimental.pallas import tpu as pltpu

_BLOCK_ROWS = 4000


def _body(x_ref, t_ref, w0_ref, w1_ref, w2_ref, b_ref, o_ref):
    x = x_ref[:]
    d0 = w0_ref.shape[0]
    d1 = w1_ref.shape[0]
    y0 = jnp.dot(x[:, :d0], w0_ref[:], preferred_element_type=jnp.float32)
    y1 = jnp.dot(x[:, :d1], w1_ref[:], preferred_element_type=jnp.float32)
    y2 = jnp.dot(x, w2_ref[:], preferred_element_type=jnp.float32)
    b = b_ref[:]
    t = t_ref[:]
    y0 = y0 + b[0:1, :]
    y1 = y1 + b[1:2, :]
    y2 = y2 + b[2:3, :]
    out = jnp.where(t == 0, y0, jnp.where(t == 1, y1, y2))
    o_ref[:] = jnp.where(t >= 3, 0.0, out)


@functools.partial(jax.jit, static_argnames=("interpret",))
def _run(feats, ntypes, w0, w1, w2, b_all, interpret=False):
    n, d = feats.shape
    p = w2.shape[1]
    r = _BLOCK_ROWS
    grid = (n // r,)
    return pl.pallas_call(
        _body,
        grid=grid,
        in_specs=[
            pl.BlockSpec((r, d), lambda i: (i, 0)),
            pl.BlockSpec((r, 1), lambda i: (i, 0)),
            pl.BlockSpec(w0.shape, lambda i: (0, 0)),
            pl.BlockSpec(w1.shape, lambda i: (0, 0)),
            pl.BlockSpec(w2.shape, lambda i: (0, 0)),
            pl.BlockSpec((3, p), lambda i: (0, 0)),
        ],
        out_specs=pl.BlockSpec((r, p), lambda i: (i, 0)),
        out_shape=jax.ShapeDtypeStruct((n, p), feats.dtype),
        compiler_params=pltpu.CompilerParams(
            dimension_semantics=("arbitrary",),
        ),
        interpret=interpret,
    )(feats, ntypes, w0, w1, w2, b_all)


def kernel(feats, ntypes, W_device, b_device, W_inst, b_inst, W_net, b_net):
    b_all = jnp.stack([b_device, b_inst, b_net], axis=0)
    t2d = ntypes.reshape(-1, 1)
    return _run(feats, t2d, W_device, W_inst, W_net, b_all)


# speedup vs baseline: 1.1224x; 1.0113x over previous
import functools

import jax
import jax.numpy as jnp
from jax.experimental import pallas as pl
from jax.experimental.pallas import tpu as pltpu

_BLOCK_ROWS = 10000


def _body(x_ref, t_ref, w0_ref, w1_ref, w2_ref, b_ref, o_ref):
    x = x_ref[:]
    d0 = w0_ref.shape[0]
    d1 = w1_ref.shape[0]
    y0 = jnp.dot(x[:, :d0], w0_ref[:], preferred_element_type=jnp.float32)
    y1 = jnp.dot(x[:, :d1], w1_ref[:], preferred_element_type=jnp.float32)
    y2 = jnp.dot(x, w2_ref[:], preferred_element_type=jnp.float32)
    b = b_ref[:]
    t = t_ref[:]
    y0 = y0 + b[0:1, :]
    y1 = y1 + b[1:2, :]
    y2 = y2 + b[2:3, :]
    out = jnp.where(t == 0, y0, jnp.where(t == 1, y1, y2))
    o_ref[:] = jnp.where(t >= 3, 0.0, out)


@functools.partial(jax.jit, static_argnames=("interpret",))
def _run(feats, ntypes, w0, w1, w2, b_all, interpret=False):
    n, d = feats.shape
    p = w2.shape[1]
    r = _BLOCK_ROWS
    grid = (n // r,)
    return pl.pallas_call(
        _body,
        grid=grid,
        in_specs=[
            pl.BlockSpec((r, d), lambda i: (i, 0)),
            pl.BlockSpec((r, 1), lambda i: (i, 0)),
            pl.BlockSpec(w0.shape, lambda i: (0, 0)),
            pl.BlockSpec(w1.shape, lambda i: (0, 0)),
            pl.BlockSpec(w2.shape, lambda i: (0, 0)),
            pl.BlockSpec((3, p), lambda i: (0, 0)),
        ],
        out_specs=pl.BlockSpec((r, p), lambda i: (i, 0)),
        out_shape=jax.ShapeDtypeStruct((n, p), feats.dtype),
        compiler_params=pltpu.CompilerParams(
            dimension_semantics=("arbitrary",),
        ),
        interpret=interpret,
    )(feats, ntypes, w0, w1, w2, b_all)


def kernel(feats, ntypes, W_device, b_device, W_inst, b_inst, W_net, b_net):
    b_all = jnp.stack([b_device, b_inst, b_net], axis=0)
    t2d = ntypes.reshape(-1, 1)
    return _run(feats, t2d, W_device, W_inst, W_net, b_all)
